# baseline (speedup 1.0000x reference)
import functools

import jax
import jax.numpy as jnp
from jax import lax
from jax.experimental import pallas as pl
from jax.experimental.pallas import tpu as pltpu
from jax.experimental.pallas import tpu_sc as plsc

NEG = -1e30


def _deg_kernel(np_, nwd):
    mesh = plsc.VectorSubcoreMesh(core_axis_name="c", subcore_axis_name="s")
    chunk = np_ // 16

    @functools.partial(
        pl.kernel,
        out_type=jax.ShapeDtypeStruct((2, np_), jnp.float32),
        mesh=mesh,
        scratch_types=[
            pltpu.VMEM((nwd, 128), jnp.int32),
            pltpu.VMEM((128,), jnp.float32),
            pltpu.VMEM((chunk,), jnp.float32),
            pltpu.VMEM_SHARED((np_,), jnp.float32),
        ],
        compiler_params=pltpu.CompilerParams(use_tc_tiling_on_sc=False),
    )
    def k(cols_hbm, deg_hbm, colv, ones, degv, deg_sp):
        c = lax.axis_index("c")
        s = lax.axis_index("s")
        w = c * 16 + s
        base = s * chunk

        def fill_ones(i, _):
            ones[pl.ds(i * 16, 16)] = jnp.full((16,), 1.0, jnp.float32)
            return 0

        lax.fori_loop(0, 8, fill_ones, 0)

        def fill_zero(i, _):
            degv[pl.ds(i * 16, 16)] = jnp.zeros((16,), jnp.float32)
            return 0

        lax.fori_loop(0, chunk // 16, fill_zero, 0)
        pltpu.sync_copy(degv, deg_sp.at[pl.ds(base, chunk)])
        pltpu.sync_copy(cols_hbm.at[w], colv)
        plsc.subcore_barrier()

        def scat(i, _):
            pltpu.sync_copy(ones, deg_sp.at[colv.at[i]], add=True)
            return 0

        lax.fori_loop(0, nwd, scat, 0)
        plsc.subcore_barrier()
        pltpu.sync_copy(deg_sp.at[pl.ds(base, chunk)], degv)
        pltpu.sync_copy(degv, deg_hbm.at[c].at[pl.ds(base, chunk)])

    return k


def _layer_kernel(np_, nwin, ksteps, f):
    mesh = plsc.VectorSubcoreMesh(core_axis_name="c", subcore_axis_name="s")
    chunk = np_ // 16
    nbuf = 4
    look = 2
    wsz = 128
    assert nwin % nbuf == 0
    fdt = jnp.float32

    @functools.partial(
        pl.kernel,
        out_type=(
            jax.ShapeDtypeStruct((2, np_, f), fdt),
            jax.ShapeDtypeStruct((2, np_, f), fdt),
            jax.ShapeDtypeStruct((2, np_, f), fdt),
        ),
        mesh=mesh,
        scratch_types=[
            pltpu.VMEM((nwin, wsz), jnp.int32),
            pltpu.VMEM((nwin, wsz), jnp.int32),
            [pltpu.VMEM((wsz, f), fdt) for _ in range(4)],
            [pltpu.SemaphoreType.DMA for _ in range(4)],
            [pltpu.SemaphoreType.DMA for _ in range(4)],
            pltpu.VMEM((64, f), fdt),
            pltpu.VMEM((64, f), fdt),
            pltpu.VMEM((64, f), fdt),
            pltpu.VMEM((16,), fdt),
            pltpu.VMEM_SHARED((np_, f), fdt),
        ],
        compiler_params=pltpu.CompilerParams(use_tc_tiling_on_sc=False),
    )
    def k(s_in, rows_hbm, cols_hbm, dis2x_hbm, att_hbm,
          aggs_out, sa_hbm, sb_hbm,
          rows_v, cols_v, bufs, gsems, ssems,
          accv, disv, aggv, attk_v, acc_sp):
        gbuf = bufs[0]
        c = lax.axis_index("c")
        s = lax.axis_index("s")
        base = s * chunk

        def fill_zero(i, _):
            for q in range(f // 16):
                accv[i, pl.ds(q * 16, 16)] = jnp.zeros((16,), fdt)
            return 0

        lax.fori_loop(0, 64, fill_zero, 0)

        def init_sub(i, _):
            off = base + i * 64
            sl = pl.ds(off, 64)
            pltpu.sync_copy(accv, acc_sp.at[sl])
            pltpu.sync_copy(accv, aggs_out.at[c].at[sl])
            pltpu.sync_copy(s_in.at[c].at[sl], gbuf.at[pl.ds(0, 64)])
            pltpu.sync_copy(gbuf.at[pl.ds(0, 64)], sb_hbm.at[c].at[sl])
            return 0

        lax.fori_loop(0, chunk // 64, init_sub, 0)
        pltpu.sync_copy(rows_hbm.at[s], rows_v)
        pltpu.sync_copy(cols_hbm.at[s], cols_v)
        plsc.subcore_barrier()

        def step(src_hbm, dst_hbm, kidx):
            for b in range(look):
                pltpu.async_copy(
                    src_hbm.at[c].at[cols_v.at[b]], bufs[b], gsems[b])

            def win(j, _):
                w0 = nbuf * j
                for b in range(nbuf):
                    w = w0 + b
                    pltpu.make_async_copy(
                        src_hbm.at[c].at[cols_v.at[w]], bufs[b],
                        gsems[b]).wait()
                    pltpu.async_copy(
                        bufs[b], acc_sp.at[rows_v.at[w]], ssems[b],
                        add=True)
                    b2 = (b + look) % nbuf

                    @pl.when(w + look < nwin)
                    def _():
                        @pl.when(w + look >= nbuf)
                        def _():
                            pltpu.make_async_copy(
                                bufs[b2], acc_sp.at[rows_v.at[0]],
                                ssems[b2]).wait()

                        pltpu.async_copy(
                            src_hbm.at[c].at[cols_v.at[w + look]],
                            bufs[b2], gsems[b2])
                return 0

            lax.fori_loop(0, nwin // nbuf, win, 0)
            for b in range(nbuf):
                pltpu.make_async_copy(
                    bufs[b], acc_sp.at[rows_v.at[0]], ssems[b]).wait()
            plsc.subcore_barrier()
            pltpu.sync_copy(att_hbm.at[kidx], attk_v)
            attk = attk_v[...]

            def wb(i, _):
                off = base + i * 64
                sl = pl.ds(off, 64)
                pltpu.sync_copy(acc_sp.at[sl], accv)
                pltpu.sync_copy(dis2x_hbm.at[sl], disv)
                pltpu.sync_copy(aggs_out.at[c].at[sl], aggv)

                def row(r, _):
                    for q in range(f // 16):
                        qs = pl.ds(q * 16, 16)
                        a = accv[r, qs] * disv[r, qs]
                        gbuf[r, qs] = a
                        aggv[r, qs] = aggv[r, qs] + attk * a
                        accv[r, qs] = jnp.zeros((16,), fdt)
                    return 0

                lax.fori_loop(0, 64, row, 0)
                pltpu.sync_copy(aggv, aggs_out.at[c].at[sl])
                pltpu.sync_copy(
                    gbuf.at[pl.ds(0, 64)], dst_hbm.at[c].at[sl])
                pltpu.sync_copy(accv, acc_sp.at[sl])
                return 0

            lax.fori_loop(0, chunk // 64, wb, 0)
            plsc.subcore_barrier()

        def pair(j, _):
            step(sb_hbm, sa_hbm, 2 * j)
            step(sa_hbm, sb_hbm, 2 * j + 1)
            return 0

        lax.fori_loop(0, ksteps // 2, pair, 0)

    return k


def _tc_prep(np_, n, d, h, bm):
    grid = np_ // bm

    def body(x_ref, w_ref, b_ref, d0_ref, d1_ref,
             h_ref, s_ref, dis_ref, dinv_ref, dis2x_ref):
        i = pl.program_id(0)
        rows = i * bm + lax.broadcasted_iota(jnp.int32, (bm, 1), 0)
        valid = rows < n
        deg = d0_ref[...] + d1_ref[...]
        pos = jnp.logical_and(valid, deg > 0)
        dis = jnp.where(pos, lax.rsqrt(jnp.maximum(deg, 1.0)), 0.0)
        dinv = jnp.where(pos, jnp.sqrt(jnp.maximum(deg, 1.0)), 0.0)
        hm = jnp.dot(x_ref[...], w_ref[...],
                     preferred_element_type=jnp.float32) + b_ref[...]
        h_ref[...] = hm
        s_ref[...] = hm * dis
        dis_ref[...] = dis
        dinv_ref[...] = dinv
        dis2x_ref[...] = jnp.broadcast_to(dis * dis, (bm, 64))

    return pl.pallas_call(
        body,
        grid=(grid,),
        in_specs=[
            pl.BlockSpec((bm, d), lambda i: (i, 0)),
            pl.BlockSpec((d, h), lambda i: (0, 0)),
            pl.BlockSpec((1, h), lambda i: (0, 0)),
            pl.BlockSpec((bm, 1), lambda i: (i, 0)),
            pl.BlockSpec((bm, 1), lambda i: (i, 0)),
        ],
        out_specs=[
            pl.BlockSpec((bm, h), lambda i: (i, 0)),
            pl.BlockSpec((bm, h), lambda i: (i, 0)),
            pl.BlockSpec((bm, 1), lambda i: (i, 0)),
            pl.BlockSpec((bm, 1), lambda i: (i, 0)),
            pl.BlockSpec((bm, 64), lambda i: (i, 0)),
        ],
        out_shape=[
            jax.ShapeDtypeStruct((np_, h), jnp.float32),
            jax.ShapeDtypeStruct((np_, h), jnp.float32),
            jax.ShapeDtypeStruct((np_, 1), jnp.float32),
            jax.ShapeDtypeStruct((np_, 1), jnp.float32),
            jax.ShapeDtypeStruct((np_, 64), jnp.float32),
        ],
    )


def _tc_mid(np_, h, bm):
    grid = np_ // bm

    def body(h0_ref, ag_ref, dis_ref, dinv_ref, w_ref, b_ref, a0_ref,
             h1_ref, s1_ref):
        a0 = a0_ref[0, 0]
        aggx = a0 * h0_ref[...] + dinv_ref[...] * ag_ref[...]
        g = jnp.maximum(aggx, 0.0)
        hm = jnp.dot(g, w_ref[...],
                     preferred_element_type=jnp.float32) + b_ref[...]
        h1_ref[...] = hm
        s1_ref[...] = hm * dis_ref[...]

    return pl.pallas_call(
        body,
        grid=(grid,),
        in_specs=[
            pl.BlockSpec((bm, h), lambda i: (i, 0)),
            pl.BlockSpec((bm, h), lambda i: (i, 0)),
            pl.BlockSpec((bm, 1), lambda i: (i, 0)),
            pl.BlockSpec((bm, 1), lambda i: (i, 0)),
            pl.BlockSpec((h, h), lambda i: (0, 0)),
            pl.BlockSpec((1, h), lambda i: (0, 0)),
            pl.BlockSpec((1, 1), lambda i: (0, 0), memory_space=pltpu.SMEM),
        ],
        out_specs=[
            pl.BlockSpec((bm, h), lambda i: (i, 0)),
            pl.BlockSpec((bm, h), lambda i: (i, 0)),
        ],
        out_shape=[
            jax.ShapeDtypeStruct((np_, h), jnp.float32),
            jax.ShapeDtypeStruct((np_, h), jnp.float32),
        ],
    )


def _tc_final(np_, h, bm):
    grid = np_ // bm

    def body(h1_ref, ag_ref, dinv_ref, w_ref, b_ref, a0_ref, o_ref):
        a0 = a0_ref[0, 0]
        aggx = a0 * h1_ref[...] + dinv_ref[...] * ag_ref[...]
        g = jnp.maximum(aggx, 0.0)
        o = jnp.dot(g, w_ref[...],
                    preferred_element_type=jnp.float32) + b_ref[...]
        m = jnp.max(o, axis=1, keepdims=True)
        p = o - m
        lse = jnp.log(jnp.sum(jnp.exp(p), axis=1, keepdims=True))
        o_ref[...] = p - lse

    return pl.pallas_call(
        body,
        grid=(grid,),
        in_specs=[
            pl.BlockSpec((bm, h), lambda i: (i, 0)),
            pl.BlockSpec((bm, h), lambda i: (i, 0)),
            pl.BlockSpec((bm, 1), lambda i: (i, 0)),
            pl.BlockSpec((h, h), lambda i: (0, 0)),
            pl.BlockSpec((1, h), lambda i: (0, 0)),
            pl.BlockSpec((1, 1), lambda i: (0, 0), memory_space=pltpu.SMEM),
        ],
        out_specs=pl.BlockSpec((bm, h), lambda i: (i, 0)),
        out_shape=jax.ShapeDtypeStruct((np_, h), jnp.float32),
    )


def kernel(x, edge_index, W0, b0, W1, b1, Wout, bout, att):
    n, d = x.shape
    h = W0.shape[1]
    c = Wout.shape[1]
    e = edge_index.shape[1]
    ksteps = att.shape[1] - 1
    f = h // 2

    np_ = ((n + 64 + 2047) // 2048) * 2048
    npad = np_ - n
    row = edge_index[0]
    col = edge_index[1]

    ept = e // 16
    nwin = -(-ept // 128)
    nwin = -(-nwin // 4) * 4
    pad = nwin * 128 - ept
    spread = jnp.arange(max(pad, 1), dtype=jnp.int32) % max(npad - 8, 1) + n
    rows_t = jnp.concatenate(
        [row.reshape(16, ept), jnp.broadcast_to(spread[:pad], (16, pad))], 1
    ).reshape(16, nwin, 128)
    cols_t = jnp.concatenate(
        [col.reshape(16, ept), jnp.broadcast_to(spread[:pad], (16, pad))], 1
    ).reshape(16, nwin, 128)

    epw = e // 32
    nwd = -(-epw // 128)
    padd = nwd * 128 - epw
    spread_d = jnp.arange(max(padd, 1), dtype=jnp.int32) % max(npad - 8, 1) + n
    cols_d = jnp.concatenate(
        [col.reshape(32, epw), jnp.broadcast_to(spread_d[:padd], (32, padd))], 1
    ).reshape(32, nwd, 128)

    deg_p = _deg_kernel(np_, nwd)(cols_d)

    xp = jnp.pad(x, ((0, npad), (0, 0)))
    bm = 512
    h0, s0f, dis, dinv, dis2x = _tc_prep(np_, n, d, h, bm)(
        xp, W0, b0.reshape(1, h),
        deg_p[0].reshape(np_, 1), deg_p[1].reshape(np_, 1))

    layer = _layer_kernel(np_, nwin, ksteps, f)

    def run_layer(sf, li):
        s_stack = jnp.stack([sf[:, :f], sf[:, f:]])
        att_e = jnp.broadcast_to(att[li, 1:, None], (ksteps, 16))
        att_e = att_e.astype(jnp.float32)
        aggs, _, _ = layer(s_stack, rows_t, cols_t, dis2x, att_e)
        return jnp.concatenate([aggs[0], aggs[1]], axis=1)

    aggs0 = run_layer(s0f, 0)
    a00 = att[0:1, 0:1].astype(jnp.float32)
    h1, s1f = _tc_mid(np_, h, bm)(
        h0, aggs0, dis, dinv, W1, b1.reshape(1, h), a00)

    aggs1 = run_layer(s1f, 1)
    a10 = att[1:2, 0:1].astype(jnp.float32)
    woutp = jnp.pad(Wout, ((0, 0), (0, h - c)))
    boutp = jnp.concatenate(
        [bout, jnp.full((h - c,), NEG, jnp.float32)]).reshape(1, h)
    out = _tc_final(np_, h, bm)(h1, aggs1, dinv, woutp, boutp, a10)
    return out[:n, :c]


# speedup vs baseline: 1.5421x; 1.5421x over previous
import functools

import jax
import jax.numpy as jnp
from jax import lax
from jax.experimental import pallas as pl
from jax.experimental.pallas import tpu as pltpu
from jax.experimental.pallas import tpu_sc as plsc

NEG = -1e30


def _deg_kernel(np_, nwd):
    mesh = plsc.VectorSubcoreMesh(core_axis_name="c", subcore_axis_name="s")
    chunk = np_ // 16

    @functools.partial(
        pl.kernel,
        out_type=jax.ShapeDtypeStruct((2, np_), jnp.float32),
        mesh=mesh,
        scratch_types=[
            pltpu.VMEM((nwd, 128), jnp.int32),
            pltpu.VMEM((128,), jnp.float32),
            pltpu.VMEM((chunk,), jnp.float32),
            pltpu.VMEM_SHARED((np_,), jnp.float32),
        ],
        compiler_params=pltpu.CompilerParams(use_tc_tiling_on_sc=False),
    )
    def k(cols_hbm, deg_hbm, colv, ones, degv, deg_sp):
        c = lax.axis_index("c")
        s = lax.axis_index("s")
        w = c * 16 + s
        base = s * chunk

        def fill_ones(i, _):
            ones[pl.ds(i * 16, 16)] = jnp.full((16,), 1.0, jnp.float32)
            return 0

        lax.fori_loop(0, 8, fill_ones, 0)

        def fill_zero(i, _):
            degv[pl.ds(i * 16, 16)] = jnp.zeros((16,), jnp.float32)
            return 0

        lax.fori_loop(0, chunk // 16, fill_zero, 0)
        pltpu.sync_copy(degv, deg_sp.at[pl.ds(base, chunk)])
        pltpu.sync_copy(cols_hbm.at[w], colv)
        plsc.subcore_barrier()

        def scat(i, _):
            pltpu.sync_copy(ones, deg_sp.at[colv.at[i]], add=True)
            return 0

        lax.fori_loop(0, nwd, scat, 0)
        plsc.subcore_barrier()
        pltpu.sync_copy(deg_sp.at[pl.ds(base, chunk)], degv)
        pltpu.sync_copy(degv, deg_hbm.at[c].at[pl.ds(base, chunk)])

    return k


def _layer_kernel(np_, nwin, ksteps, f):
    mesh = plsc.VectorSubcoreMesh(core_axis_name="c", subcore_axis_name="s")
    chunk = np_ // 16
    nbuf = 6
    wsz = 128
    assert nwin % nbuf == 0
    fdt = jnp.float32
    bdt = jnp.bfloat16

    @functools.partial(
        pl.kernel,
        out_type=(
            jax.ShapeDtypeStruct((2, np_, f), fdt),
            jax.ShapeDtypeStruct((2, np_, f), bdt),
            jax.ShapeDtypeStruct((2, np_, f), bdt),
        ),
        mesh=mesh,
        scratch_types=[
            pltpu.VMEM((nwin, wsz), jnp.int32),
            pltpu.VMEM((nwin, wsz), jnp.int32),
            [pltpu.VMEM((wsz, f), bdt) for _ in range(6)],
            [pltpu.SemaphoreType.DMA for _ in range(6)],
            pltpu.VMEM((64, f), bdt),
            pltpu.VMEM((64, f), fdt),
            pltpu.VMEM((64, f), fdt),
            pltpu.VMEM((16,), fdt),
            pltpu.VMEM_SHARED((np_, f), bdt),
        ],
        compiler_params=pltpu.CompilerParams(
            use_tc_tiling_on_sc=False, needs_layout_passes=False),
    )
    def k(s_in, rows_hbm, cols_hbm, dis2x_hbm, att_hbm,
          aggs_out, sa_hbm, sb_hbm,
          rows_v, cols_v, bufs, gsems,
          accv, disv, aggv, attk_v, acc_sp):
        gbuf = bufs[0]
        c = lax.axis_index("c")
        s = lax.axis_index("s")
        base = s * chunk

        def fill_zero(i, _):
            for q in range(f // 32):
                accv[i, pl.ds(q * 32, 32)] = jnp.zeros((32,), bdt)
            for q in range(f // 16):
                aggv[i, pl.ds(q * 16, 16)] = jnp.zeros((16,), fdt)
            return 0

        lax.fori_loop(0, 64, fill_zero, 0)

        def init_sub(i, _):
            off = base + i * 64
            sl = pl.ds(off, 64)
            pltpu.sync_copy(accv, acc_sp.at[sl])
            pltpu.sync_copy(aggv, aggs_out.at[c].at[sl])
            pltpu.sync_copy(s_in.at[c].at[sl], gbuf.at[pl.ds(0, 64)])
            pltpu.sync_copy(gbuf.at[pl.ds(0, 64)], sb_hbm.at[c].at[sl])
            return 0

        lax.fori_loop(0, chunk // 64, init_sub, 0)
        pltpu.sync_copy(rows_hbm.at[s], rows_v)
        pltpu.sync_copy(cols_hbm.at[s], cols_v)
        plsc.subcore_barrier()

        def step(src_hbm, dst_hbm, kidx):
            for b in range(nbuf):
                pltpu.async_copy(
                    src_hbm.at[c].at[cols_v.at[b]], bufs[b], gsems[b])

            def win(j, _):
                w0 = nbuf * j
                for b in range(nbuf):
                    w = w0 + b
                    pltpu.make_async_copy(
                        src_hbm.at[c].at[cols_v.at[w]], bufs[b],
                        gsems[b]).wait()
                    pltpu.sync_copy(
                        bufs[b], acc_sp.at[rows_v.at[w]], add=True)

                    @pl.when(w + nbuf < nwin)
                    def _():
                        pltpu.async_copy(
                            src_hbm.at[c].at[cols_v.at[w + nbuf]], bufs[b],
                            gsems[b])
                return 0

            lax.fori_loop(0, nwin // nbuf, win, 0)
            plsc.subcore_barrier()
            pltpu.sync_copy(att_hbm.at[kidx], attk_v)
            attk = attk_v[...]

            def wb(i, _):
                off = base + i * 64
                sl = pl.ds(off, 64)
                pltpu.sync_copy(acc_sp.at[sl], accv)
                pltpu.sync_copy(dis2x_hbm.at[sl], disv)
                pltpu.sync_copy(aggs_out.at[c].at[sl], aggv)

                def row(r, _):
                    d = disv[r, pl.ds(0, 16)]
                    for q in range(f // 32):
                        q32 = pl.ds(q * 32, 32)
                        a, b = plsc.unpack(
                            accv[r, q32],
                            format=plsc.PackFormat.INTERLEAVED)
                        sa = a * d
                        sb = b * d
                        gbuf[r, q32] = plsc.pack(
                            sa, sb, format=plsc.PackFormat.INTERLEAVED)
                        qa = pl.ds(q * 32, 16)
                        qb = pl.ds(q * 32 + 16, 16)
                        aggv[r, qa] = aggv[r, qa] + attk * sa
                        aggv[r, qb] = aggv[r, qb] + attk * sb
                        accv[r, q32] = jnp.zeros((32,), bdt)
                    return 0

                lax.fori_loop(0, 64, row, 0)
                pltpu.sync_copy(aggv, aggs_out.at[c].at[sl])
                pltpu.sync_copy(
                    gbuf.at[pl.ds(0, 64)], dst_hbm.at[c].at[sl])
                pltpu.sync_copy(accv, acc_sp.at[sl])
                return 0

            lax.fori_loop(0, chunk // 64, wb, 0)
            plsc.subcore_barrier()

        def pair(j, _):
            step(sb_hbm, sa_hbm, 2 * j)
            step(sa_hbm, sb_hbm, 2 * j + 1)
            return 0

        lax.fori_loop(0, ksteps // 2, pair, 0)

    return k


def _tc_prep(np_, n, d, h, bm):
    grid = np_ // bm

    def body(x_ref, w_ref, b_ref, d0_ref, d1_ref,
             h_ref, s_ref, dis_ref, dinv_ref, dis2x_ref):
        i = pl.program_id(0)
        rows = i * bm + lax.broadcasted_iota(jnp.int32, (bm, 1), 0)
        valid = rows < n
        deg = d0_ref[...] + d1_ref[...]
        pos = jnp.logical_and(valid, deg > 0)
        dis = jnp.where(pos, lax.rsqrt(jnp.maximum(deg, 1.0)), 0.0)
        dinv = jnp.where(pos, jnp.sqrt(jnp.maximum(deg, 1.0)), 0.0)
        hm = jnp.dot(x_ref[...], w_ref[...],
                     preferred_element_type=jnp.float32) + b_ref[...]
        h_ref[...] = hm
        s_ref[...] = hm * dis
        dis_ref[...] = dis
        dinv_ref[...] = dinv
        dis2x_ref[...] = jnp.broadcast_to(dis * dis, (bm, 64))

    return pl.pallas_call(
        body,
        grid=(grid,),
        in_specs=[
            pl.BlockSpec((bm, d), lambda i: (i, 0)),
            pl.BlockSpec((d, h), lambda i: (0, 0)),
            pl.BlockSpec((1, h), lambda i: (0, 0)),
            pl.BlockSpec((bm, 1), lambda i: (i, 0)),
            pl.BlockSpec((bm, 1), lambda i: (i, 0)),
        ],
        out_specs=[
            pl.BlockSpec((bm, h), lambda i: (i, 0)),
            pl.BlockSpec((bm, h), lambda i: (i, 0)),
            pl.BlockSpec((bm, 1), lambda i: (i, 0)),
            pl.BlockSpec((bm, 1), lambda i: (i, 0)),
            pl.BlockSpec((bm, 64), lambda i: (i, 0)),
        ],
        out_shape=[
            jax.ShapeDtypeStruct((np_, h), jnp.float32),
            jax.ShapeDtypeStruct((np_, h), jnp.float32),
            jax.ShapeDtypeStruct((np_, 1), jnp.float32),
            jax.ShapeDtypeStruct((np_, 1), jnp.float32),
            jax.ShapeDtypeStruct((np_, 64), jnp.float32),
        ],
    )


def _tc_mid(np_, h, bm):
    grid = np_ // bm

    def body(h0_ref, ag_ref, dis_ref, dinv_ref, w_ref, b_ref, a0_ref,
             h1_ref, s1_ref):
        a0 = a0_ref[0, 0]
        aggx = a0 * h0_ref[...] + dinv_ref[...] * ag_ref[...]
        g = jnp.maximum(aggx, 0.0)
        hm = jnp.dot(g, w_ref[...],
                     preferred_element_type=jnp.float32) + b_ref[...]
        h1_ref[...] = hm
        s1_ref[...] = hm * dis_ref[...]

    return pl.pallas_call(
        body,
        grid=(grid,),
        in_specs=[
            pl.BlockSpec((bm, h), lambda i: (i, 0)),
            pl.BlockSpec((bm, h), lambda i: (i, 0)),
            pl.BlockSpec((bm, 1), lambda i: (i, 0)),
            pl.BlockSpec((bm, 1), lambda i: (i, 0)),
            pl.BlockSpec((h, h), lambda i: (0, 0)),
            pl.BlockSpec((1, h), lambda i: (0, 0)),
            pl.BlockSpec((1, 1), lambda i: (0, 0), memory_space=pltpu.SMEM),
        ],
        out_specs=[
            pl.BlockSpec((bm, h), lambda i: (i, 0)),
            pl.BlockSpec((bm, h), lambda i: (i, 0)),
        ],
        out_shape=[
            jax.ShapeDtypeStruct((np_, h), jnp.float32),
            jax.ShapeDtypeStruct((np_, h), jnp.float32),
        ],
    )


def _tc_final(np_, h, bm):
    grid = np_ // bm

    def body(h1_ref, ag_ref, dinv_ref, w_ref, b_ref, a0_ref, o_ref):
        a0 = a0_ref[0, 0]
        aggx = a0 * h1_ref[...] + dinv_ref[...] * ag_ref[...]
        g = jnp.maximum(aggx, 0.0)
        o = jnp.dot(g, w_ref[...],
                    preferred_element_type=jnp.float32) + b_ref[...]
        m = jnp.max(o, axis=1, keepdims=True)
        p = o - m
        lse = jnp.log(jnp.sum(jnp.exp(p), axis=1, keepdims=True))
        o_ref[...] = p - lse

    return pl.pallas_call(
        body,
        grid=(grid,),
        in_specs=[
            pl.BlockSpec((bm, h), lambda i: (i, 0)),
            pl.BlockSpec((bm, h), lambda i: (i, 0)),
            pl.BlockSpec((bm, 1), lambda i: (i, 0)),
            pl.BlockSpec((h, h), lambda i: (0, 0)),
            pl.BlockSpec((1, h), lambda i: (0, 0)),
            pl.BlockSpec((1, 1), lambda i: (0, 0), memory_space=pltpu.SMEM),
        ],
        out_specs=pl.BlockSpec((bm, h), lambda i: (i, 0)),
        out_shape=jax.ShapeDtypeStruct((np_, h), jnp.float32),
    )


def kernel(x, edge_index, W0, b0, W1, b1, Wout, bout, att):
    n, d = x.shape
    h = W0.shape[1]
    c = Wout.shape[1]
    e = edge_index.shape[1]
    ksteps = att.shape[1] - 1
    f = h // 2

    np_ = ((n + 64 + 2047) // 2048) * 2048
    npad = np_ - n
    row = edge_index[0]
    col = edge_index[1]

    ept = e // 16
    nwin = -(-ept // 128)
    nwin = -(-nwin // 6) * 6
    pad = nwin * 128 - ept
    spread = jnp.arange(max(pad, 1), dtype=jnp.int32) % max(npad - 8, 1) + n
    rows_t = jnp.concatenate(
        [row.reshape(16, ept), jnp.broadcast_to(spread[:pad], (16, pad))], 1
    ).reshape(16, nwin, 128)
    cols_t = jnp.concatenate(
        [col.reshape(16, ept), jnp.broadcast_to(spread[:pad], (16, pad))], 1
    ).reshape(16, nwin, 128)

    epw = e // 32
    nwd = -(-epw // 128)
    padd = nwd * 128 - epw
    spread_d = jnp.arange(max(padd, 1), dtype=jnp.int32) % max(npad - 8, 1) + n
    cols_d = jnp.concatenate(
        [col.reshape(32, epw), jnp.broadcast_to(spread_d[:padd], (32, padd))], 1
    ).reshape(32, nwd, 128)

    deg_p = _deg_kernel(np_, nwd)(cols_d)

    xp = jnp.pad(x, ((0, npad), (0, 0)))
    bm = 512
    h0, s0f, dis, dinv, dis2x = _tc_prep(np_, n, d, h, bm)(
        xp, W0, b0.reshape(1, h),
        deg_p[0].reshape(np_, 1), deg_p[1].reshape(np_, 1))

    layer = _layer_kernel(np_, nwin, ksteps, f)

    o = jnp.arange(f)
    unperm = 32 * (o // 32) + (o % 2) * 16 + (o % 32) // 2

    def run_layer(sf, li):
        sb16 = sf.astype(jnp.bfloat16)
        s_stack = jnp.stack([sb16[:, :f], sb16[:, f:]])
        att_e = jnp.broadcast_to(att[li, 1:, None], (ksteps, 16))
        att_e = att_e.astype(jnp.float32)
        aggs, _, _ = layer(s_stack, rows_t, cols_t, dis2x, att_e)
        a0 = jnp.take(aggs[0], unperm, axis=1)
        a1 = jnp.take(aggs[1], unperm, axis=1)
        return jnp.concatenate([a0, a1], axis=1)

    aggs0 = run_layer(s0f, 0)
    a00 = att[0:1, 0:1].astype(jnp.float32)
    h1, s1f = _tc_mid(np_, h, bm)(
        h0, aggs0, dis, dinv, W1, b1.reshape(1, h), a00)

    aggs1 = run_layer(s1f, 1)
    a10 = att[1:2, 0:1].astype(jnp.float32)
    woutp = jnp.pad(Wout, ((0, 0), (0, h - c)))
    boutp = jnp.concatenate(
        [bout, jnp.full((h - c,), NEG, jnp.float32)]).reshape(1, h)
    out = _tc_final(np_, h, bm)(h1, aggs1, dinv, woutp, boutp, a10)
    return out[:n, :c]


# speedup vs baseline: 1.5823x; 1.0261x over previous
import functools

import jax
import jax.numpy as jnp
from jax import lax
from jax.experimental import pallas as pl
from jax.experimental.pallas import tpu as pltpu
from jax.experimental.pallas import tpu_sc as plsc

NEG = -1e30


def _deg_kernel(np_, nwd):
    mesh = plsc.VectorSubcoreMesh(core_axis_name="c", subcore_axis_name="s")
    chunk = np_ // 16

    @functools.partial(
        pl.kernel,
        out_type=jax.ShapeDtypeStruct((2, np_), jnp.float32),
        mesh=mesh,
        scratch_types=[
            pltpu.VMEM((nwd, 128), jnp.int32),
            pltpu.VMEM((128,), jnp.float32),
            pltpu.VMEM((chunk,), jnp.float32),
            pltpu.VMEM_SHARED((np_,), jnp.float32),
        ],
        compiler_params=pltpu.CompilerParams(use_tc_tiling_on_sc=False),
    )
    def k(cols_hbm, deg_hbm, colv, ones, degv, deg_sp):
        c = lax.axis_index("c")
        s = lax.axis_index("s")
        w = c * 16 + s
        base = s * chunk

        def fill_ones(i, _):
            ones[pl.ds(i * 16, 16)] = jnp.full((16,), 1.0, jnp.float32)
            return 0

        lax.fori_loop(0, 8, fill_ones, 0)

        def fill_zero(i, _):
            degv[pl.ds(i * 16, 16)] = jnp.zeros((16,), jnp.float32)
            return 0

        lax.fori_loop(0, chunk // 16, fill_zero, 0)
        pltpu.sync_copy(degv, deg_sp.at[pl.ds(base, chunk)])
        pltpu.sync_copy(cols_hbm.at[w], colv)
        plsc.subcore_barrier()

        def scat(i, _):
            pltpu.sync_copy(ones, deg_sp.at[colv.at[i]], add=True)
            return 0

        lax.fori_loop(0, nwd, scat, 0)
        plsc.subcore_barrier()
        pltpu.sync_copy(deg_sp.at[pl.ds(base, chunk)], degv)
        pltpu.sync_copy(degv, deg_hbm.at[c].at[pl.ds(base, chunk)])

    return k


def _layer_kernel(np_, nwin, ksteps, f):
    mesh = plsc.VectorSubcoreMesh(core_axis_name="c", subcore_axis_name="s")
    chunk = np_ // 16
    nbuf = 8
    wsz = 128
    assert nwin % nbuf == 0
    fdt = jnp.float32
    bdt = jnp.bfloat16

    @functools.partial(
        pl.kernel,
        out_type=(
            jax.ShapeDtypeStruct((2, np_, f), fdt),
            jax.ShapeDtypeStruct((2, np_, f), bdt),
            jax.ShapeDtypeStruct((2, np_, f), bdt),
        ),
        mesh=mesh,
        scratch_types=[
            pltpu.VMEM((nwin, wsz), jnp.int32),
            pltpu.VMEM((nwin, wsz), jnp.int32),
            [pltpu.VMEM((wsz, f), bdt) for _ in range(8)],
            [pltpu.SemaphoreType.DMA for _ in range(8)],
            pltpu.VMEM((64, f), bdt),
            pltpu.VMEM((64, f), fdt),
            pltpu.VMEM((64, f), fdt),
            pltpu.VMEM((16,), fdt),
            pltpu.VMEM_SHARED((np_, f), bdt),
        ],
        compiler_params=pltpu.CompilerParams(
            use_tc_tiling_on_sc=False, needs_layout_passes=False),
    )
    def k(s_in, rows_hbm, cols_hbm, dis2x_hbm, att_hbm,
          aggs_out, sa_hbm, sb_hbm,
          rows_v, cols_v, bufs, gsems,
          accv, disv, aggv, attk_v, acc_sp):
        gbuf = bufs[0]
        c = lax.axis_index("c")
        s = lax.axis_index("s")
        base = s * chunk

        def fill_zero(i, _):
            for q in range(f // 32):
                accv[i, pl.ds(q * 32, 32)] = jnp.zeros((32,), bdt)
            for q in range(f // 16):
                aggv[i, pl.ds(q * 16, 16)] = jnp.zeros((16,), fdt)
            return 0

        lax.fori_loop(0, 64, fill_zero, 0)

        def init_sub(i, _):
            off = base + i * 64
            sl = pl.ds(off, 64)
            pltpu.sync_copy(accv, acc_sp.at[sl])
            pltpu.sync_copy(aggv, aggs_out.at[c].at[sl])
            pltpu.sync_copy(s_in.at[c].at[sl], gbuf.at[pl.ds(0, 64)])
            pltpu.sync_copy(gbuf.at[pl.ds(0, 64)], sb_hbm.at[c].at[sl])
            return 0

        lax.fori_loop(0, chunk // 64, init_sub, 0)
        pltpu.sync_copy(rows_hbm.at[s], rows_v)
        pltpu.sync_copy(cols_hbm.at[s], cols_v)
        plsc.subcore_barrier()

        def step(src_hbm, dst_hbm, kidx):
            for b in range(nbuf):
                pltpu.async_copy(
                    src_hbm.at[c].at[cols_v.at[b]], bufs[b], gsems[b])

            def win(j, _):
                w0 = nbuf * j
                for b in range(nbuf):
                    w = w0 + b
                    pltpu.make_async_copy(
                        src_hbm.at[c].at[cols_v.at[w]], bufs[b],
                        gsems[b]).wait()
                    pltpu.sync_copy(
                        bufs[b], acc_sp.at[rows_v.at[w]], add=True)

                    @pl.when(w + nbuf < nwin)
                    def _():
                        pltpu.async_copy(
                            src_hbm.at[c].at[cols_v.at[w + nbuf]], bufs[b],
                            gsems[b])
                return 0

            lax.fori_loop(0, nwin // nbuf, win, 0)
            plsc.subcore_barrier()
            pltpu.sync_copy(att_hbm.at[kidx], attk_v)
            attk = attk_v[...]

            def wb(i, _):
                off = base + i * 64
                sl = pl.ds(off, 64)
                pltpu.sync_copy(acc_sp.at[sl], accv)
                pltpu.sync_copy(dis2x_hbm.at[sl], disv)
                pltpu.sync_copy(aggs_out.at[c].at[sl], aggv)

                def row(r, _):
                    d = disv[r, pl.ds(0, 16)]
                    for q in range(f // 32):
                        q32 = pl.ds(q * 32, 32)
                        a, b = plsc.unpack(
                            accv[r, q32],
                            format=plsc.PackFormat.INTERLEAVED)
                        sa = a * d
                        sb = b * d
                        gbuf[r, q32] = plsc.pack(
                            sa, sb, format=plsc.PackFormat.INTERLEAVED)
                        qa = pl.ds(q * 32, 16)
                        qb = pl.ds(q * 32 + 16, 16)
                        aggv[r, qa] = aggv[r, qa] + attk * sa
                        aggv[r, qb] = aggv[r, qb] + attk * sb
                        accv[r, q32] = jnp.zeros((32,), bdt)
                    return 0

                lax.fori_loop(0, 64, row, 0)
                pltpu.sync_copy(aggv, aggs_out.at[c].at[sl])
                pltpu.sync_copy(
                    gbuf.at[pl.ds(0, 64)], dst_hbm.at[c].at[sl])
                pltpu.sync_copy(accv, acc_sp.at[sl])
                return 0

            lax.fori_loop(0, chunk // 64, wb, 0)
            plsc.subcore_barrier()

        def pair(j, _):
            step(sb_hbm, sa_hbm, 2 * j)
            step(sa_hbm, sb_hbm, 2 * j + 1)
            return 0

        lax.fori_loop(0, ksteps // 2, pair, 0)

    return k


def _tc_prep(np_, n, d, h, bm):
    grid = np_ // bm

    def body(x_ref, w_ref, b_ref, d0_ref, d1_ref,
             h_ref, s_ref, dis_ref, dinv_ref, dis2x_ref):
        i = pl.program_id(0)
        rows = i * bm + lax.broadcasted_iota(jnp.int32, (bm, 1), 0)
        valid = rows < n
        deg = d0_ref[...] + d1_ref[...]
        pos = jnp.logical_and(valid, deg > 0)
        dis = jnp.where(pos, lax.rsqrt(jnp.maximum(deg, 1.0)), 0.0)
        dinv = jnp.where(pos, jnp.sqrt(jnp.maximum(deg, 1.0)), 0.0)
        hm = jnp.dot(x_ref[...], w_ref[...],
                     preferred_element_type=jnp.float32) + b_ref[...]
        h_ref[...] = hm
        s_ref[...] = hm * dis
        dis_ref[...] = dis
        dinv_ref[...] = dinv
        dis2x_ref[...] = jnp.broadcast_to(dis * dis, (bm, 64))

    return pl.pallas_call(
        body,
        grid=(grid,),
        in_specs=[
            pl.BlockSpec((bm, d), lambda i: (i, 0)),
            pl.BlockSpec((d, h), lambda i: (0, 0)),
            pl.BlockSpec((1, h), lambda i: (0, 0)),
            pl.BlockSpec((bm, 1), lambda i: (i, 0)),
            pl.BlockSpec((bm, 1), lambda i: (i, 0)),
        ],
        out_specs=[
            pl.BlockSpec((bm, h), lambda i: (i, 0)),
            pl.BlockSpec((bm, h), lambda i: (i, 0)),
            pl.BlockSpec((bm, 1), lambda i: (i, 0)),
            pl.BlockSpec((bm, 1), lambda i: (i, 0)),
            pl.BlockSpec((bm, 64), lambda i: (i, 0)),
        ],
        out_shape=[
            jax.ShapeDtypeStruct((np_, h), jnp.float32),
            jax.ShapeDtypeStruct((np_, h), jnp.float32),
            jax.ShapeDtypeStruct((np_, 1), jnp.float32),
            jax.ShapeDtypeStruct((np_, 1), jnp.float32),
            jax.ShapeDtypeStruct((np_, 64), jnp.float32),
        ],
    )


def _tc_mid(np_, h, bm):
    grid = np_ // bm

    def body(h0_ref, ag_ref, dis_ref, dinv_ref, w_ref, b_ref, a0_ref,
             h1_ref, s1_ref):
        a0 = a0_ref[0, 0]
        aggx = a0 * h0_ref[...] + dinv_ref[...] * ag_ref[...]
        g = jnp.maximum(aggx, 0.0)
        hm = jnp.dot(g, w_ref[...],
                     preferred_element_type=jnp.float32) + b_ref[...]
        h1_ref[...] = hm
        s1_ref[...] = hm * dis_ref[...]

    return pl.pallas_call(
        body,
        grid=(grid,),
        in_specs=[
            pl.BlockSpec((bm, h), lambda i: (i, 0)),
            pl.BlockSpec((bm, h), lambda i: (i, 0)),
            pl.BlockSpec((bm, 1), lambda i: (i, 0)),
            pl.BlockSpec((bm, 1), lambda i: (i, 0)),
            pl.BlockSpec((h, h), lambda i: (0, 0)),
            pl.BlockSpec((1, h), lambda i: (0, 0)),
            pl.BlockSpec((1, 1), lambda i: (0, 0), memory_space=pltpu.SMEM),
        ],
        out_specs=[
            pl.BlockSpec((bm, h), lambda i: (i, 0)),
            pl.BlockSpec((bm, h), lambda i: (i, 0)),
        ],
        out_shape=[
            jax.ShapeDtypeStruct((np_, h), jnp.float32),
            jax.ShapeDtypeStruct((np_, h), jnp.float32),
        ],
    )


def _tc_final(np_, h, bm):
    grid = np_ // bm

    def body(h1_ref, ag_ref, dinv_ref, w_ref, b_ref, a0_ref, o_ref):
        a0 = a0_ref[0, 0]
        aggx = a0 * h1_ref[...] + dinv_ref[...] * ag_ref[...]
        g = jnp.maximum(aggx, 0.0)
        o = jnp.dot(g, w_ref[...],
                    preferred_element_type=jnp.float32) + b_ref[...]
        m = jnp.max(o, axis=1, keepdims=True)
        p = o - m
        lse = jnp.log(jnp.sum(jnp.exp(p), axis=1, keepdims=True))
        o_ref[...] = p - lse

    return pl.pallas_call(
        body,
        grid=(grid,),
        in_specs=[
            pl.BlockSpec((bm, h), lambda i: (i, 0)),
            pl.BlockSpec((bm, h), lambda i: (i, 0)),
            pl.BlockSpec((bm, 1), lambda i: (i, 0)),
            pl.BlockSpec((h, h), lambda i: (0, 0)),
            pl.BlockSpec((1, h), lambda i: (0, 0)),
            pl.BlockSpec((1, 1), lambda i: (0, 0), memory_space=pltpu.SMEM),
        ],
        out_specs=pl.BlockSpec((bm, h), lambda i: (i, 0)),
        out_shape=jax.ShapeDtypeStruct((np_, h), jnp.float32),
    )


def kernel(x, edge_index, W0, b0, W1, b1, Wout, bout, att):
    n, d = x.shape
    h = W0.shape[1]
    c = Wout.shape[1]
    e = edge_index.shape[1]
    ksteps = att.shape[1] - 1
    f = h // 2

    np_ = ((n + 64 + 2047) // 2048) * 2048
    npad = np_ - n
    row = edge_index[0]
    col = edge_index[1]

    ept = e // 16
    nwin = -(-ept // 128)
    nwin = -(-nwin // 8) * 8
    pad = nwin * 128 - ept
    spread = jnp.arange(max(pad, 1), dtype=jnp.int32) % max(npad - 8, 1) + n
    rows_t = jnp.concatenate(
        [row.reshape(16, ept), jnp.broadcast_to(spread[:pad], (16, pad))], 1
    ).reshape(16, nwin, 128)
    cols_t = jnp.concatenate(
        [col.reshape(16, ept), jnp.broadcast_to(spread[:pad], (16, pad))], 1
    ).reshape(16, nwin, 128)

    epw = e // 32
    nwd = -(-epw // 128)
    padd = nwd * 128 - epw
    spread_d = jnp.arange(max(padd, 1), dtype=jnp.int32) % max(npad - 8, 1) + n
    cols_d = jnp.concatenate(
        [col.reshape(32, epw), jnp.broadcast_to(spread_d[:padd], (32, padd))], 1
    ).reshape(32, nwd, 128)

    deg_p = _deg_kernel(np_, nwd)(cols_d)

    xp = jnp.pad(x, ((0, npad), (0, 0)))
    bm = 512
    h0, s0f, dis, dinv, dis2x = _tc_prep(np_, n, d, h, bm)(
        xp, W0, b0.reshape(1, h),
        deg_p[0].reshape(np_, 1), deg_p[1].reshape(np_, 1))

    layer = _layer_kernel(np_, nwin, ksteps, f)

    o = jnp.arange(f)
    unperm = 32 * (o // 32) + (o % 2) * 16 + (o % 32) // 2

    def run_layer(sf, li):
        sb16 = sf.astype(jnp.bfloat16)
        s_stack = jnp.stack([sb16[:, :f], sb16[:, f:]])
        att_e = jnp.broadcast_to(att[li, 1:, None], (ksteps, 16))
        att_e = att_e.astype(jnp.float32)
        aggs, _, _ = layer(s_stack, rows_t, cols_t, dis2x, att_e)
        a0 = jnp.take(aggs[0], unperm, axis=1)
        a1 = jnp.take(aggs[1], unperm, axis=1)
        return jnp.concatenate([a0, a1], axis=1)

    aggs0 = run_layer(s0f, 0)
    a00 = att[0:1, 0:1].astype(jnp.float32)
    h1, s1f = _tc_mid(np_, h, bm)(
        h0, aggs0, dis, dinv, W1, b1.reshape(1, h), a00)

    aggs1 = run_layer(s1f, 1)
    a10 = att[1:2, 0:1].astype(jnp.float32)
    woutp = jnp.pad(Wout, ((0, 0), (0, h - c)))
    boutp = jnp.concatenate(
        [bout, jnp.full((h - c,), NEG, jnp.float32)]).reshape(1, h)
    out = _tc_final(np_, h, bm)(h1, aggs1, dinv, woutp, boutp, a10)
    return out[:n, :c]
